# baseline (speedup 1.0000x reference)
import jax
import jax.numpy as jnp
from jax.experimental import pallas as pl
from jax.experimental.pallas import tpu as pltpu

_L = 1_000_000
_BIG = 65_536
_SMALL = 8_192
_TINY = 2_048
_K = 24
_DI = 12


def _block_widths(i):
    odd = _L - 4 * _TINY - 7 * _SMALL - 14 * _BIG
    if i == 0:
        return [_TINY] * 4 + [_SMALL] * 7 + [_BIG] * 14 + [odd]
    if i == 3:
        return [_BIG] * 14 + [_SMALL] * 7 + [_TINY] * 4 + [odd]
    return [_BIG] * 15 + [_L - 15 * _BIG]


_CHUNKS = []
_ODD = []
for _i in range(4):
    _off = 0
    for _w in _block_widths(_i):
        (_CHUNKS if _w % 128 == 0 else _ODD).append((_i, _off, _w))
        _off += _w
    assert _off == _L
assert len(_ODD) == 4 and all(_w == _ODD[0][2] for (_, _, _w) in _ODD)
_ODD_W = _ODD[0][2]


def _copy_body(src, dst, bufs, tbufs, in_sems, out_sems, tin_sems, tout_sems):
    s3 = src.reshape(4, 8, _L)
    d3 = dst.reshape(4, 8, _L)
    n_chunks = len(_CHUNKS)

    def in_copy(c):
        i, off, w = _CHUNKS[c]
        return pltpu.make_async_copy(
            s3.at[i, :, pl.ds(off, w)],
            bufs.at[c % _K, :, pl.ds(0, w)],
            in_sems.at[c % _K],
        )

    def out_copy(c):
        i, off, w = _CHUNKS[c]
        return pltpu.make_async_copy(
            bufs.at[c % _K, :, pl.ds(0, w)],
            d3.at[i, :, pl.ds(off, w)],
            out_sems.at[c % _K],
        )

    def tin_copy(j):
        i, off, w = _ODD[j]
        return pltpu.make_async_copy(
            s3.at[i, :, pl.ds(off, w)], tbufs.at[j], tin_sems.at[j]
        )

    def tout_copy(j):
        i, off, w = _ODD[j]
        return pltpu.make_async_copy(
            tbufs.at[j], d3.at[i, :, pl.ds(off, w)], tout_sems.at[j]
        )

    for c in range(min(_DI, n_chunks)):
        in_copy(c).start()
    for j in range(4):
        tin_copy(j).start()
    waited = set()
    for c in range(n_chunks):
        in_copy(c).wait()
        out_copy(c).start()
        p = c + _DI
        if p < n_chunks:
            if p >= _K:
                out_copy(p - _K).wait()
                waited.add(p - _K)
            in_copy(p).start()
        if c == n_chunks // 2:
            for j in range(4):
                tin_copy(j).wait()
                tout_copy(j).start()
    for c in range(n_chunks):
        if c not in waited:
            out_copy(c).wait()
    for j in range(4):
        tout_copy(j).wait()


def kernel(embeds):
    t = embeds.T
    out = pl.pallas_call(
        _copy_body,
        out_shape=jax.ShapeDtypeStruct(t.shape, t.dtype),
        in_specs=[pl.BlockSpec(memory_space=pltpu.MemorySpace.HBM)],
        out_specs=pl.BlockSpec(memory_space=pltpu.MemorySpace.HBM),
        scratch_shapes=[
            pltpu.VMEM((_K, 8, _BIG), jnp.float32),
            pltpu.VMEM((4, 8, _ODD_W), jnp.float32),
            pltpu.SemaphoreType.DMA((_K,)),
            pltpu.SemaphoreType.DMA((_K,)),
            pltpu.SemaphoreType.DMA((4,)),
            pltpu.SemaphoreType.DMA((4,)),
        ],
    )(t)
    return out.T
